# Initial kernel scaffold; baseline (speedup 1.0000x reference)
import functools

import jax
import jax.numpy as jnp
from jax.experimental import pallas as pl
from jax.experimental.pallas import tpu as pltpu

EPS = 1e-5
LANE = 128

_CP = pltpu.CompilerParams(
    dimension_semantics=("parallel",),
    vmem_limit_bytes=64 * 1024 * 1024,
)


def _rup(x, m):
    return (x + m - 1) // m * m


def _tile(m, target):
    if m <= target:
        return m
    for t in range(target, 7, -1):
        if m % t == 0 and t % 8 == 0:
            return t
    return m


def _pad_last(a, c):
    pad = c - a.shape[-1]
    if pad == 0:
        return a
    return jnp.pad(a, [(0, 0)] * (a.ndim - 1) + [(0, pad)])


def _stats(yf):
    return jnp.concatenate(
        [jnp.sum(yf, 0, keepdims=True), jnp.sum(yf * yf, 0, keepdims=True)], 0)


def _fold(stats, count, gamma, beta):
    s = jnp.sum(stats[:, 0, :], axis=0)
    ss = jnp.sum(stats[:, 1, :], axis=0)
    mean = s / count
    var = jnp.maximum(ss / count - mean * mean, 0.0)
    scale = gamma * jax.lax.rsqrt(var + EPS)
    return scale, beta - mean * scale


def _mm_stats_kernel(x_ref, w_ref, y_ref, st_ref):
    y = jnp.dot(x_ref[...], w_ref[...], preferred_element_type=jnp.float32)
    yb = y.astype(jnp.bfloat16)
    y_ref[...] = yb
    st_ref[0] = _stats(yb.astype(jnp.float32))


def _tap(z, ky, kx, ho, wo):
    if ky == 0:
        rs = z[1:2 * ho - 2:2]
    elif ky == 1:
        rs = z[0:2 * ho - 1:2]
    else:
        rs = z[1:2 * ho:2]
    if kx == 0:
        cs = rs[:, 1:2 * wo - 2:2]
    elif kx == 1:
        cs = rs[:, 0:2 * wo - 1:2]
    else:
        cs = rs[:, 1:2 * wo:2]
    padr = 1 if ky == 0 else 0
    padc = 1 if kx == 0 else 0
    if padr or padc:
        cs = jnp.pad(cs, ((padr, 0), (padc, 0), (0, 0)))
    return cs


def _conv2_kernel(y1_ref, s1_ref, b1_ref, w2_ref, y2_ref, st_ref, z_ref,
                  *, ho, wo):
    cp = y1_ref.shape[-1]
    s1 = s1_ref[0].reshape(1, 1, cp)
    b1 = b1_ref[0].reshape(1, 1, cp)
    z_ref[...] = jnp.maximum(
        y1_ref[0].astype(jnp.float32) * s1 + b1, 0.0).astype(jnp.bfloat16)
    z = z_ref[...]
    acc = jnp.zeros((ho * wo, cp), jnp.float32)
    for ky in range(3):
        parts = [
            _tap(z, ky, kx, ho, wo).reshape(ho * wo, cp) for kx in range(3)
        ]
        wide = jnp.concatenate(parts, axis=1)
        acc = acc + jnp.dot(wide, w2_ref[ky],
                            preferred_element_type=jnp.float32)
    yb = acc.astype(jnp.bfloat16)
    y2_ref[0] = yb
    st_ref[0] = _stats(yb.astype(jnp.float32))


def _dual_mm_kernel(y2_ref, s2_ref, b2_ref, w3_ref, xs_ref, ws_ref,
                    y3_ref, st3_ref, ysc_ref, stsc_ref):
    z2 = jnp.maximum(
        y2_ref[...].astype(jnp.float32) * s2_ref[...] + b2_ref[...],
        0.0).astype(jnp.bfloat16)
    y3 = jnp.dot(z2, w3_ref[...],
                 preferred_element_type=jnp.float32).astype(jnp.bfloat16)
    y3_ref[...] = y3
    st3_ref[0] = _stats(y3.astype(jnp.float32))
    ysc = jnp.dot(xs_ref[...], ws_ref[...],
                  preferred_element_type=jnp.float32).astype(jnp.bfloat16)
    ysc_ref[...] = ysc
    stsc_ref[0] = _stats(ysc.astype(jnp.float32))


def _final_kernel(y3_ref, ysc_ref, s3_ref, b3_ref, ssc_ref, bsc_ref, o_ref):
    a = y3_ref[...].astype(jnp.float32) * s3_ref[...] + b3_ref[...]
    b = ysc_ref[...].astype(jnp.float32) * ssc_ref[...] + bsc_ref[...]
    o_ref[...] = jnp.maximum(a + b, 0.0)


def kernel(x, w1, g1, b1, w2, g2, b2, w3, g3, b3, ws, gs, bs):
    N, Cin, H, W = x.shape
    planes = w1.shape[0]
    cout = w3.shape[0]
    cp = _rup(planes, LANE)
    cpo = _rup(cout, LANE)
    Ho, Wo = (H - 1) // 2 + 1, (W - 1) // 2 + 1
    M1, M2 = N * H * W, N * Ho * Wo
    bf = jnp.bfloat16
    f32 = jnp.float32

    w1m = _pad_last(w1[:, :, 0, 0].T, cp).astype(bf)
    w2t = jnp.transpose(w2, (2, 3, 1, 0))
    w2m = jnp.zeros((3, 3 * cp, cp), f32)
    for ky in range(3):
        for kx in range(3):
            w2m = w2m.at[ky, kx * cp:kx * cp + planes, :planes].set(
                w2t[ky, kx])
    w2m = w2m.astype(bf)
    w3m = jnp.zeros((cp, cpo), f32).at[:planes, :cout].set(
        w3[:, :, 0, 0].T).astype(bf)
    wsm = _pad_last(ws[:, :, 0, 0].T, cpo).astype(bf)
    g1p, b1p = _pad_last(g1, cp), _pad_last(b1, cp)
    g2p, b2p = _pad_last(g2, cp), _pad_last(b2, cp)
    g3p, b3p = _pad_last(g3, cpo), _pad_last(b3, cpo)
    gsp, bsp = _pad_last(gs, cpo), _pad_last(bs, cpo)

    xb = jnp.transpose(x, (0, 2, 3, 1)).astype(bf)
    x2d = xb.reshape(M1, Cin)

    TM1 = _tile(M1, 512)
    gr1 = M1 // TM1
    y1, st1 = pl.pallas_call(
        _mm_stats_kernel,
        grid=(gr1,),
        in_specs=[pl.BlockSpec((TM1, Cin), lambda i: (i, 0)),
                  pl.BlockSpec((Cin, cp), lambda i: (0, 0))],
        out_specs=[pl.BlockSpec((TM1, cp), lambda i: (i, 0)),
                   pl.BlockSpec((1, 2, cp), lambda i: (i, 0, 0))],
        out_shape=[jax.ShapeDtypeStruct((M1, cp), bf),
                   jax.ShapeDtypeStruct((gr1, 2, cp), f32)],
        compiler_params=_CP,
    )(x2d, w1m)
    s1, h1 = _fold(st1, M1, g1p, b1p)

    k2 = functools.partial(_conv2_kernel, ho=Ho, wo=Wo)
    y2, st2 = pl.pallas_call(
        k2,
        grid=(N,),
        in_specs=[pl.BlockSpec((1, H, W, cp), lambda n: (n, 0, 0, 0)),
                  pl.BlockSpec((1, cp), lambda n: (0, 0)),
                  pl.BlockSpec((1, cp), lambda n: (0, 0)),
                  pl.BlockSpec((3, 3 * cp, cp), lambda n: (0, 0, 0))],
        out_specs=[pl.BlockSpec((1, Ho * Wo, cp), lambda n: (n, 0, 0)),
                   pl.BlockSpec((1, 2, cp), lambda n: (n, 0, 0))],
        out_shape=[jax.ShapeDtypeStruct((N, Ho * Wo, cp), bf),
                   jax.ShapeDtypeStruct((N, 2, cp), f32)],
        scratch_shapes=[pltpu.VMEM((H, W, cp), bf)],
        compiler_params=_CP,
    )(y1.reshape(N, H, W, cp), s1.reshape(1, cp), h1.reshape(1, cp), w2m)
    s2, h2 = _fold(st2, M2, g2p, b2p)

    xs2d = xb[:, ::2, ::2, :].reshape(M2, Cin)
    TM2 = _tile(M2, 512)
    gr2 = M2 // TM2
    y3, st3, ysc, stsc = pl.pallas_call(
        _dual_mm_kernel,
        grid=(gr2,),
        in_specs=[pl.BlockSpec((TM2, cp), lambda i: (i, 0)),
                  pl.BlockSpec((1, cp), lambda i: (0, 0)),
                  pl.BlockSpec((1, cp), lambda i: (0, 0)),
                  pl.BlockSpec((cp, cpo), lambda i: (0, 0)),
                  pl.BlockSpec((TM2, Cin), lambda i: (i, 0)),
                  pl.BlockSpec((Cin, cpo), lambda i: (0, 0))],
        out_specs=[pl.BlockSpec((TM2, cpo), lambda i: (i, 0)),
                   pl.BlockSpec((1, 2, cpo), lambda i: (i, 0, 0)),
                   pl.BlockSpec((TM2, cpo), lambda i: (i, 0)),
                   pl.BlockSpec((1, 2, cpo), lambda i: (i, 0, 0))],
        out_shape=[jax.ShapeDtypeStruct((M2, cpo), bf),
                   jax.ShapeDtypeStruct((gr2, 2, cpo), f32),
                   jax.ShapeDtypeStruct((M2, cpo), bf),
                   jax.ShapeDtypeStruct((gr2, 2, cpo), f32)],
        compiler_params=_CP,
    )(y2.reshape(M2, cp), s2.reshape(1, cp), h2.reshape(1, cp), w3m,
      xs2d, wsm)
    s3, h3 = _fold(st3, M2, g3p, b3p)
    ssc, hsc = _fold(stsc, M2, gsp, bsp)

    out2d = pl.pallas_call(
        _final_kernel,
        grid=(gr2,),
        in_specs=[pl.BlockSpec((TM2, cpo), lambda i: (i, 0)),
                  pl.BlockSpec((TM2, cpo), lambda i: (i, 0)),
                  pl.BlockSpec((1, cpo), lambda i: (0, 0)),
                  pl.BlockSpec((1, cpo), lambda i: (0, 0)),
                  pl.BlockSpec((1, cpo), lambda i: (0, 0)),
                  pl.BlockSpec((1, cpo), lambda i: (0, 0))],
        out_specs=pl.BlockSpec((TM2, cpo), lambda i: (i, 0)),
        out_shape=jax.ShapeDtypeStruct((M2, cpo), f32),
        compiler_params=_CP,
    )(y3, ysc, s3.reshape(1, cpo), h3.reshape(1, cpo),
      ssc.reshape(1, cpo), hsc.reshape(1, cpo))

    out = out2d[:, :cout].reshape(N, Ho, Wo, cout)
    return jnp.transpose(out, (0, 3, 1, 2))


# speedup vs baseline: 3.9975x; 3.9975x over previous
import functools

import jax
import jax.numpy as jnp
from jax.experimental import pallas as pl
from jax.experimental.pallas import tpu as pltpu

EPS = 1e-5
LANE = 128

_CP = pltpu.CompilerParams(
    dimension_semantics=("parallel",),
    vmem_limit_bytes=64 * 1024 * 1024,
)


def _rup(x, m):
    return (x + m - 1) // m * m


def _tile(m, target):
    if m <= target:
        return m
    for t in range(target, 7, -1):
        if m % t == 0 and t % 8 == 0:
            return t
    return m


def _pad_last(a, c):
    pad = c - a.shape[-1]
    if pad == 0:
        return a
    return jnp.pad(a, [(0, 0)] * (a.ndim - 1) + [(0, pad)])


def _stats(yf):
    return jnp.concatenate(
        [jnp.sum(yf, 0, keepdims=True), jnp.sum(yf * yf, 0, keepdims=True)], 0)


def _fold(stats, count, gamma, beta):
    s = jnp.sum(stats[:, 0, :], axis=0)
    ss = jnp.sum(stats[:, 1, :], axis=0)
    mean = s / count
    var = jnp.maximum(ss / count - mean * mean, 0.0)
    scale = gamma * jax.lax.rsqrt(var + EPS)
    return scale, beta - mean * scale


def _mm_stats_kernel(x_ref, w_ref, y_ref, st_ref):
    y = jnp.dot(x_ref[...], w_ref[...], preferred_element_type=jnp.float32)
    yb = y.astype(jnp.bfloat16)
    y_ref[...] = yb
    st_ref[0] = _stats(yb.astype(jnp.float32))


def _conv2_kernel(y1_ref, s1_ref, b1_ref, w2_ref, y2_ref, st_ref, z_ref,
                  *, ho, wo):
    cp = y1_ref.shape[-1]
    hw = ho * wo
    s1 = s1_ref[...]
    b1 = b1_ref[...]
    for p in range(4):
        z_ref[p] = jnp.maximum(
            y1_ref[0, p].astype(jnp.float32) * s1 + b1, 0.0
        ).astype(jnp.bfloat16)
    colmask = (jax.lax.broadcasted_iota(jnp.int32, (hw, 1), 0) % wo) != 0
    acc = jnp.zeros((hw, w2_ref.shape[-1]), jnp.float32)
    for ky in range(3):
        parts = []
        for kx in range(3):
            a, b = (ky + 1) % 2, (kx + 1) % 2
            base = z_ref[a * 2 + b]
            shift = (wo if ky == 0 else 0) + (1 if kx == 0 else 0)
            if shift:
                base = jnp.concatenate(
                    [jnp.zeros((shift, cp), base.dtype), base[:hw - shift]],
                    axis=0)
            if kx == 0:
                base = jnp.where(colmask, base, jnp.bfloat16(0))
            parts.append(base)
        wide = jnp.concatenate(parts, axis=1)
        acc = acc + jnp.dot(wide, w2_ref[ky],
                            preferred_element_type=jnp.float32)
    yb = acc.astype(jnp.bfloat16)
    y2_ref[0] = yb
    st_ref[0] = _stats(yb.astype(jnp.float32))


def _dual_mm_kernel(y2_ref, s2_ref, b2_ref, w3_ref, xs_ref, ws_ref,
                    y3_ref, st3_ref, ysc_ref, stsc_ref):
    z2 = jnp.maximum(
        y2_ref[...].astype(jnp.float32) * s2_ref[...] + b2_ref[...],
        0.0).astype(jnp.bfloat16)
    y3 = jnp.dot(z2, w3_ref[...],
                 preferred_element_type=jnp.float32).astype(jnp.bfloat16)
    y3_ref[...] = y3
    st3_ref[0] = _stats(y3.astype(jnp.float32))
    ysc = jnp.dot(xs_ref[...], ws_ref[...],
                  preferred_element_type=jnp.float32).astype(jnp.bfloat16)
    ysc_ref[...] = ysc
    stsc_ref[0] = _stats(ysc.astype(jnp.float32))


def _final_kernel(y3_ref, ysc_ref, s3_ref, b3_ref, ssc_ref, bsc_ref, o_ref):
    a = y3_ref[...].astype(jnp.float32) * s3_ref[...] + b3_ref[...]
    b = ysc_ref[...].astype(jnp.float32) * ssc_ref[...] + bsc_ref[...]
    o_ref[...] = jnp.maximum(a + b, 0.0)


def kernel(x, w1, g1, b1, w2, g2, b2, w3, g3, b3, ws, gs, bs):
    N, Cin, H, W = x.shape
    planes = w1.shape[0]
    cout = w3.shape[0]
    cp = _rup(planes, LANE)
    cpo = _rup(cout, LANE)
    Ho, Wo = (H - 1) // 2 + 1, (W - 1) // 2 + 1
    M1, M2 = N * H * W, N * Ho * Wo
    bf = jnp.bfloat16
    f32 = jnp.float32

    w1m = _pad_last(w1[:, :, 0, 0].T, cp).astype(bf)
    w2t = jnp.transpose(w2, (2, 3, 1, 0))
    w2m = jnp.zeros((3, 3 * cp, cp), f32)
    for ky in range(3):
        for kx in range(3):
            w2m = w2m.at[ky, kx * cp:kx * cp + planes, :planes].set(
                w2t[ky, kx])
    w2m = w2m.astype(bf)
    w3m = jnp.zeros((cp, cpo), f32).at[:planes, :cout].set(
        w3[:, :, 0, 0].T).astype(bf)
    wsm = _pad_last(ws[:, :, 0, 0].T, cpo).astype(bf)
    g1p, b1p = _pad_last(g1, cp), _pad_last(b1, cp)
    g2p, b2p = _pad_last(g2, cp), _pad_last(b2, cp)
    g3p, b3p = _pad_last(g3, cpo), _pad_last(b3, cpo)
    gsp, bsp = _pad_last(gs, cpo), _pad_last(bs, cpo)

    x2d = jnp.transpose(
        x.reshape(N, Cin, Ho, 2, Wo, 2), (0, 3, 5, 2, 4, 1)
    ).astype(bf).reshape(M1, Cin)

    TM1 = _tile(M1, 512)
    gr1 = M1 // TM1
    y1, st1 = pl.pallas_call(
        _mm_stats_kernel,
        grid=(gr1,),
        in_specs=[pl.BlockSpec((TM1, Cin), lambda i: (i, 0)),
                  pl.BlockSpec((Cin, cp), lambda i: (0, 0))],
        out_specs=[pl.BlockSpec((TM1, cp), lambda i: (i, 0)),
                   pl.BlockSpec((1, 2, cp), lambda i: (i, 0, 0))],
        out_shape=[jax.ShapeDtypeStruct((M1, cp), bf),
                   jax.ShapeDtypeStruct((gr1, 2, cp), f32)],
        compiler_params=_CP,
    )(x2d, w1m)
    s1, h1 = _fold(st1, M1, g1p, b1p)

    hw = Ho * Wo
    k2 = functools.partial(_conv2_kernel, ho=Ho, wo=Wo)
    y2, st2 = pl.pallas_call(
        k2,
        grid=(N,),
        in_specs=[pl.BlockSpec((1, 4, hw, cp), lambda n: (n, 0, 0, 0)),
                  pl.BlockSpec((1, cp), lambda n: (0, 0)),
                  pl.BlockSpec((1, cp), lambda n: (0, 0)),
                  pl.BlockSpec((3, 3 * cp, cp), lambda n: (0, 0, 0))],
        out_specs=[pl.BlockSpec((1, hw, cp), lambda n: (n, 0, 0)),
                   pl.BlockSpec((1, 2, cp), lambda n: (n, 0, 0))],
        out_shape=[jax.ShapeDtypeStruct((N, hw, cp), bf),
                   jax.ShapeDtypeStruct((N, 2, cp), f32)],
        scratch_shapes=[pltpu.VMEM((4, hw, cp), bf)],
        compiler_params=_CP,
    )(y1.reshape(N, 4, hw, cp), s1.reshape(1, cp), h1.reshape(1, cp), w2m)
    s2, h2 = _fold(st2, M2, g2p, b2p)

    TM2 = hw
    gr2 = M2 // TM2
    y3, st3, ysc, stsc = pl.pallas_call(
        _dual_mm_kernel,
        grid=(gr2,),
        in_specs=[pl.BlockSpec((TM2, cp), lambda i: (i, 0)),
                  pl.BlockSpec((1, cp), lambda i: (0, 0)),
                  pl.BlockSpec((1, cp), lambda i: (0, 0)),
                  pl.BlockSpec((cp, cpo), lambda i: (0, 0)),
                  pl.BlockSpec((TM2, Cin), lambda i: (4 * i, 0)),
                  pl.BlockSpec((Cin, cpo), lambda i: (0, 0))],
        out_specs=[pl.BlockSpec((TM2, cpo), lambda i: (i, 0)),
                   pl.BlockSpec((1, 2, cpo), lambda i: (i, 0, 0)),
                   pl.BlockSpec((TM2, cpo), lambda i: (i, 0)),
                   pl.BlockSpec((1, 2, cpo), lambda i: (i, 0, 0))],
        out_shape=[jax.ShapeDtypeStruct((M2, cpo), bf),
                   jax.ShapeDtypeStruct((gr2, 2, cpo), f32),
                   jax.ShapeDtypeStruct((M2, cpo), bf),
                   jax.ShapeDtypeStruct((gr2, 2, cpo), f32)],
        compiler_params=_CP,
    )(y2.reshape(M2, cp), s2.reshape(1, cp), h2.reshape(1, cp), w3m,
      x2d, wsm)
    s3, h3 = _fold(st3, M2, g3p, b3p)
    ssc, hsc = _fold(stsc, M2, gsp, bsp)

    out2d = pl.pallas_call(
        _final_kernel,
        grid=(gr2,),
        in_specs=[pl.BlockSpec((TM2, cpo), lambda i: (i, 0)),
                  pl.BlockSpec((TM2, cpo), lambda i: (i, 0)),
                  pl.BlockSpec((1, cpo), lambda i: (0, 0)),
                  pl.BlockSpec((1, cpo), lambda i: (0, 0)),
                  pl.BlockSpec((1, cpo), lambda i: (0, 0)),
                  pl.BlockSpec((1, cpo), lambda i: (0, 0))],
        out_specs=pl.BlockSpec((TM2, cpo), lambda i: (i, 0)),
        out_shape=jax.ShapeDtypeStruct((M2, cpo), f32),
        compiler_params=_CP,
    )(y3, ysc, s3.reshape(1, cpo), h3.reshape(1, cpo),
      ssc.reshape(1, cpo), hsc.reshape(1, cpo))

    out = out2d[:, :cout].reshape(N, Ho, Wo, cout)
    return jnp.transpose(out, (0, 3, 1, 2))


# speedup vs baseline: 4.9696x; 1.2432x over previous
import functools

import jax
import jax.numpy as jnp
from jax.experimental import pallas as pl
from jax.experimental.pallas import tpu as pltpu

EPS = 1e-5
LANE = 128

_CP = pltpu.CompilerParams(
    dimension_semantics=("parallel",),
    vmem_limit_bytes=64 * 1024 * 1024,
)


def _rup(x, m):
    return (x + m - 1) // m * m


def _tile(m, target):
    if m <= target:
        return m
    for t in range(target, 7, -1):
        if m % t == 0 and t % 8 == 0:
            return t
    return m


def _pad_last(a, c):
    pad = c - a.shape[-1]
    if pad == 0:
        return a
    return jnp.pad(a, [(0, 0)] * (a.ndim - 1) + [(0, pad)])


def _stats(yf):
    return jnp.concatenate(
        [jnp.sum(yf, 0, keepdims=True), jnp.sum(yf * yf, 0, keepdims=True)], 0)


def _fold(st, count, gamma, beta):
    s = jnp.sum(st[:, 0, :], axis=0, keepdims=True)
    ss = jnp.sum(st[:, 1, :], axis=0, keepdims=True)
    mean = s / count
    var = jnp.maximum(ss / count - mean * mean, 0.0)
    scale = gamma * jax.lax.rsqrt(var + EPS)
    return scale, beta - mean * scale


def _mm_stats_kernel(x_ref, w_ref, y_ref, st_ref):
    y = jnp.dot(x_ref[...], w_ref[...], preferred_element_type=jnp.float32)
    yb = y.astype(jnp.bfloat16)
    y_ref[...] = yb
    st_ref[0] = _stats(yb.astype(jnp.float32))


def _conv2_kernel(y1_ref, st1_ref, g1_ref, c1_ref, w2_ref, y2_ref, st_ref,
                  z_ref, *, ho, wo, m1):
    cp = y1_ref.shape[-1]
    hw = ho * wo
    s1, b1 = _fold(st1_ref[...], m1, g1_ref[...], c1_ref[...])
    for p in range(4):
        z_ref[p] = jnp.maximum(
            y1_ref[0, p].astype(jnp.float32) * s1 + b1, 0.0
        ).astype(jnp.bfloat16)
    colmask = (jax.lax.broadcasted_iota(jnp.int32, (hw, 1), 0) % wo) != 0
    acc = jnp.zeros((hw, w2_ref.shape[-1]), jnp.float32)
    for ky in range(3):
        parts = []
        for kx in range(3):
            a, b = (ky + 1) % 2, (kx + 1) % 2
            base = z_ref[a * 2 + b]
            shift = (wo if ky == 0 else 0) + (1 if kx == 0 else 0)
            if shift:
                base = jnp.concatenate(
                    [jnp.zeros((shift, cp), base.dtype), base[:hw - shift]],
                    axis=0)
            if kx == 0:
                base = jnp.where(colmask, base, jnp.bfloat16(0))
            parts.append(base)
        wide = jnp.concatenate(parts, axis=1)
        acc = acc + jnp.dot(wide, w2_ref[ky],
                            preferred_element_type=jnp.float32)
    yb = acc.astype(jnp.bfloat16)
    y2_ref[0] = yb
    st_ref[0] = _stats(yb.astype(jnp.float32))


def _dual_mm_kernel(y2_ref, st2_ref, g2_ref, c2_ref, w3_ref, xs_ref, ws_ref,
                    y3_ref, st3_ref, ysc_ref, stsc_ref, *, m2):
    s2, b2 = _fold(st2_ref[...], m2, g2_ref[...], c2_ref[...])
    z2 = jnp.maximum(
        y2_ref[...].astype(jnp.float32) * s2 + b2, 0.0).astype(jnp.bfloat16)
    y3 = jnp.dot(z2, w3_ref[...],
                 preferred_element_type=jnp.float32).astype(jnp.bfloat16)
    y3_ref[...] = y3
    st3_ref[0] = _stats(y3.astype(jnp.float32))
    ysc = jnp.dot(xs_ref[...], ws_ref[...],
                  preferred_element_type=jnp.float32).astype(jnp.bfloat16)
    ysc_ref[...] = ysc
    stsc_ref[0] = _stats(ysc.astype(jnp.float32))


def _final_kernel(y3_ref, ysc_ref, st3_ref, g3_ref, c3_ref,
                  stsc_ref, gs_ref, cs_ref, o_ref, *, m2):
    s3, b3 = _fold(st3_ref[...], m2, g3_ref[...], c3_ref[...])
    ss, bs_ = _fold(stsc_ref[...], m2, gs_ref[...], cs_ref[...])
    a = y3_ref[...].astype(jnp.float32) * s3 + b3
    b = ysc_ref[...].astype(jnp.float32) * ss + bs_
    o_ref[...] = jnp.maximum(a + b, 0.0)


def kernel(x, w1, g1, b1, w2, g2, b2, w3, g3, b3, ws, gs, bs):
    N, Cin, H, W = x.shape
    planes = w1.shape[0]
    cout = w3.shape[0]
    cp = _rup(planes, LANE)
    cpo = _rup(cout, LANE)
    Ho, Wo = (H - 1) // 2 + 1, (W - 1) // 2 + 1
    hw = Ho * Wo
    M1, M2 = N * H * W, N * hw
    bf = jnp.bfloat16
    f32 = jnp.float32

    w1m = _pad_last(w1[:, :, 0, 0].T, cp).astype(bf)
    w2t = jnp.transpose(w2, (2, 3, 1, 0))
    w2m = jnp.pad(
        w2t, ((0, 0), (0, 0), (0, cp - planes), (0, cp - planes))
    ).reshape(3, 3 * cp, cp).astype(bf)
    w3m = jnp.pad(
        w3[:, :, 0, 0].T, ((0, cp - planes), (0, cpo - cout))).astype(bf)
    wsm = _pad_last(ws[:, :, 0, 0].T, cpo).astype(bf)
    g1p, b1p = _pad_last(g1, cp).reshape(1, cp), _pad_last(b1, cp).reshape(1, cp)
    g2p, b2p = _pad_last(g2, cp).reshape(1, cp), _pad_last(b2, cp).reshape(1, cp)
    g3p, b3p = (_pad_last(g3, cpo).reshape(1, cpo),
                _pad_last(b3, cpo).reshape(1, cpo))
    gsp, bsp = (_pad_last(gs, cpo).reshape(1, cpo),
                _pad_last(bs, cpo).reshape(1, cpo))

    x2d = jnp.transpose(
        x.reshape(N, Cin, Ho, 2, Wo, 2), (0, 3, 5, 2, 4, 1)
    ).astype(bf).reshape(M1, Cin)

    TM1 = _tile(M1, 4 * hw)
    gr1 = M1 // TM1
    y1, st1 = pl.pallas_call(
        _mm_stats_kernel,
        grid=(gr1,),
        in_specs=[pl.BlockSpec((TM1, Cin), lambda i: (i, 0)),
                  pl.BlockSpec((Cin, cp), lambda i: (0, 0))],
        out_specs=[pl.BlockSpec((TM1, cp), lambda i: (i, 0)),
                   pl.BlockSpec((1, 2, cp), lambda i: (i, 0, 0))],
        out_shape=[jax.ShapeDtypeStruct((M1, cp), bf),
                   jax.ShapeDtypeStruct((gr1, 2, cp), f32)],
        compiler_params=_CP,
    )(x2d, w1m)

    k2 = functools.partial(_conv2_kernel, ho=Ho, wo=Wo, m1=float(M1))
    y2, st2 = pl.pallas_call(
        k2,
        grid=(N,),
        in_specs=[pl.BlockSpec((1, 4, hw, cp), lambda n: (n, 0, 0, 0)),
                  pl.BlockSpec((gr1, 2, cp), lambda n: (0, 0, 0)),
                  pl.BlockSpec((1, cp), lambda n: (0, 0)),
                  pl.BlockSpec((1, cp), lambda n: (0, 0)),
                  pl.BlockSpec((3, 3 * cp, cp), lambda n: (0, 0, 0))],
        out_specs=[pl.BlockSpec((1, hw, cp), lambda n: (n, 0, 0)),
                   pl.BlockSpec((1, 2, cp), lambda n: (n, 0, 0))],
        out_shape=[jax.ShapeDtypeStruct((N, hw, cp), bf),
                   jax.ShapeDtypeStruct((N, 2, cp), f32)],
        scratch_shapes=[pltpu.VMEM((4, hw, cp), bf)],
        compiler_params=_CP,
    )(y1.reshape(N, 4, hw, cp), st1, g1p, b1p, w2m)

    TM2 = hw
    gr2 = M2 // TM2
    k3 = functools.partial(_dual_mm_kernel, m2=float(M2))
    y3, st3, ysc, stsc = pl.pallas_call(
        k3,
        grid=(gr2,),
        in_specs=[pl.BlockSpec((TM2, cp), lambda i: (i, 0)),
                  pl.BlockSpec((N, 2, cp), lambda i: (0, 0, 0)),
                  pl.BlockSpec((1, cp), lambda i: (0, 0)),
                  pl.BlockSpec((1, cp), lambda i: (0, 0)),
                  pl.BlockSpec((cp, cpo), lambda i: (0, 0)),
                  pl.BlockSpec((TM2, Cin), lambda i: (4 * i, 0)),
                  pl.BlockSpec((Cin, cpo), lambda i: (0, 0))],
        out_specs=[pl.BlockSpec((TM2, cpo), lambda i: (i, 0)),
                   pl.BlockSpec((1, 2, cpo), lambda i: (i, 0, 0)),
                   pl.BlockSpec((TM2, cpo), lambda i: (i, 0)),
                   pl.BlockSpec((1, 2, cpo), lambda i: (i, 0, 0))],
        out_shape=[jax.ShapeDtypeStruct((M2, cpo), bf),
                   jax.ShapeDtypeStruct((gr2, 2, cpo), f32),
                   jax.ShapeDtypeStruct((M2, cpo), bf),
                   jax.ShapeDtypeStruct((gr2, 2, cpo), f32)],
        compiler_params=_CP,
    )(y2.reshape(M2, cp), st2, g2p, b2p, w3m, x2d, wsm)

    k4 = functools.partial(_final_kernel, m2=float(M2))
    out2d = pl.pallas_call(
        k4,
        grid=(gr2,),
        in_specs=[pl.BlockSpec((TM2, cpo), lambda i: (i, 0)),
                  pl.BlockSpec((TM2, cpo), lambda i: (i, 0)),
                  pl.BlockSpec((gr2, 2, cpo), lambda i: (0, 0, 0)),
                  pl.BlockSpec((1, cpo), lambda i: (0, 0)),
                  pl.BlockSpec((1, cpo), lambda i: (0, 0)),
                  pl.BlockSpec((gr2, 2, cpo), lambda i: (0, 0, 0)),
                  pl.BlockSpec((1, cpo), lambda i: (0, 0)),
                  pl.BlockSpec((1, cpo), lambda i: (0, 0))],
        out_specs=pl.BlockSpec((TM2, cpo), lambda i: (i, 0)),
        out_shape=jax.ShapeDtypeStruct((M2, cpo), f32),
        compiler_params=_CP,
    )(y3, ysc, st3, g3p, b3p, stsc, gsp, bsp)

    out = out2d[:, :cout].reshape(N, Ho, Wo, cout)
    return jnp.transpose(out, (0, 3, 1, 2))
